# baseline (speedup 1.0000x reference)
import functools

import jax
import jax.numpy as jnp
import numpy as np
from jax.experimental import pallas as pl
from jax.experimental.pallas import tpu as pltpu

C_DIM = 32
HID = 32
DIM = 3
N_BLOCKS = 5
N_CH = 3
RESO = 64
PAD = 0.1


def _lin(x, w, b=None):
    y = x @ w
    return y + b if b is not None else y


def _fc_planenet(p, pr):
    net = _lin(p, pr['pl0_w'], pr['pl0_b'])
    net = jax.nn.relu(_lin(net, pr['pl1_w'], pr['pl1_b']))
    net = jax.nn.relu(_lin(net, pr['pl2_w'], pr['pl2_b']))
    net = jax.nn.relu(_lin(net, pr['pl3_w'], pr['pl3_b']))
    net = jnp.max(net, axis=1)
    net = jax.nn.relu(_lin(net, pr['pl4_w'], pr['pl4_b']))
    net = jax.nn.relu(_lin(net, pr['pl5_w'], pr['pl5_b']))
    net = _lin(net, pr['pl6_w'], pr['pl6_b'])
    return net


def _resblock(x, pr, i):
    net = _lin(jax.nn.relu(x), pr['blk%d_fc0_w' % i], pr['blk%d_fc0_b' % i])
    dx = _lin(jax.nn.relu(net), pr['blk%d_fc1_w' % i], pr['blk%d_fc1_b' % i])
    xs = x @ pr['blk%d_sc_w' % i]
    return xs + dx


def _change_basis(plane_parameters):
    Bc, L, _ = plane_parameters.shape
    n = Bc * L
    normal = plane_parameters.reshape(n, 3)
    normal = normal / jnp.linalg.norm(normal, axis=1, keepdims=True)
    normal = normal + 1e-06
    basis_x = jnp.tile(jnp.array([1.0, 0.0, 0.0], jnp.float32), (n, 1))
    basis_y = jnp.tile(jnp.array([0.0, 1.0, 0.0], jnp.float32), (n, 1))
    basis_z = jnp.tile(jnp.array([0.0, 0.0, 1.0], jnp.float32), (n, 1))
    v = jnp.cross(basis_z, normal)
    zero = jnp.zeros((n,), jnp.float32)
    row0 = jnp.stack([zero, -v[:, 2], v[:, 1]], axis=1)
    row1 = jnp.stack([v[:, 2], zero, -v[:, 0]], axis=1)
    row2 = jnp.stack([-v[:, 1], v[:, 0], zero], axis=1)
    skew = jnp.stack([row0, row1, row2], axis=1)
    idty = jnp.tile(jnp.eye(3, dtype=jnp.float32)[None], (n, 1, 1))
    dot = (1.0 - jnp.sum(normal * basis_z, axis=1))[:, None, None]
    div = (jnp.linalg.norm(v, axis=1) ** 2)[:, None, None]
    R = idty + skew + jnp.matmul(skew, skew) * dot / div
    new_x = jnp.matmul(R, basis_x[:, :, None])
    new_y = jnp.matmul(R, basis_y[:, :, None])
    new_z = jnp.matmul(R, basis_z[:, :, None])
    new_basis = jnp.concatenate([new_x, new_y, new_z], axis=2)
    C_inv = jnp.linalg.inv(new_basis)
    b_x = jnp.abs(new_x)[:, :, 0]
    b_y = jnp.abs(new_y)[:, :, 0]
    p_dummy = jnp.ones((n, 3), jnp.float32)
    p_x = (jnp.sum(b_x * p_dummy, axis=1, keepdims=True) / jnp.sum(b_x * b_x, axis=1, keepdims=True)) * b_x
    p_y = (jnp.sum(b_y * p_dummy, axis=1, keepdims=True) / jnp.sum(b_y * b_y, axis=1, keepdims=True)) * b_y
    c_x = jnp.linalg.norm(p_x, axis=1)
    c_y = jnp.linalg.norm(p_y, axis=1)
    normalizer = jnp.tile(jnp.maximum(c_x, c_y)[:, None, None], (1, 1, 3))
    C_mat = jnp.concatenate([C_inv, normalizer], axis=1).reshape(Bc, L, 4, 3)
    return C_mat


def _normalize_dyn(p, cmat):
    C_inv = cmat[:, :3, :]
    normalizer = cmat[:, 3, 0][:, None, None]
    p_new = jnp.einsum('bij,btj->bti', C_inv, p) / normalizer
    xy = p_new[:, :, :2]
    xy = xy / (1.0 + PAD + 1e-3) + 0.5
    xy = jnp.clip(xy, 0.0, 1.0 - 1e-6)
    return xy


def _coord2index(xy):
    x = jnp.clip((xy * RESO).astype(jnp.int32), 0, RESO - 1)
    return x[:, :, 0] + RESO * x[:, :, 1]


def _pool_local(indices, c):
    Bc, Tc, Fc = c.shape
    offs = (jnp.arange(Bc, dtype=jnp.int32) * RESO * RESO)[:, None]
    out = jnp.zeros_like(c)
    flat_c = c.reshape(Bc * Tc, Fc)
    for index in indices:
        idx_flat = (index + offs).reshape(-1)
        fea = jax.ops.segment_max(flat_c, idx_flat, num_segments=Bc * RESO * RESO)
        fea = jnp.where(jnp.isneginf(fea), 0.0, fea)
        out = out + fea[idx_flat].reshape(Bc, Tc, Fc)
    return out


def _gen_plane_feat(p, c, cmat):
    Bc, Tc, _ = p.shape
    Fc = c.shape[2]
    xy = _normalize_dyn(p, cmat)
    index = _coord2index(xy)
    offs = (jnp.arange(Bc, dtype=jnp.int32) * RESO * RESO)[:, None]
    idx_flat = (index + offs).reshape(-1)
    sums = jax.ops.segment_sum(c.reshape(Bc * Tc, Fc), idx_flat, num_segments=Bc * RESO * RESO)
    cnts = jax.ops.segment_sum(jnp.ones((Bc * Tc,), jnp.float32), idx_flat, num_segments=Bc * RESO * RESO)
    mean = sums / jnp.maximum(cnts, 1.0)[:, None]
    fea = mean.reshape(Bc, RESO * RESO, Fc).transpose(0, 2, 1).reshape(Bc, Fc, RESO, RESO)
    return fea


_TBLK = 2000
HID2 = 2 * HID
DIMC = DIM


def _trunk0_body(p_ref, wpos_ref, bpos_ref, w00_ref, b00_ref, w01_ref,
                 b01_ref, wsc_ref, o_ref):
    p = p_ref[...]
    x = jnp.dot(p, wpos_ref[...], preferred_element_type=jnp.float32) + bpos_ref[...]
    net = jnp.dot(jax.nn.relu(x), w00_ref[...], preferred_element_type=jnp.float32) + b00_ref[...]
    dx = jnp.dot(jax.nn.relu(net), w01_ref[...], preferred_element_type=jnp.float32) + b01_ref[...]
    xs = jnp.dot(x, wsc_ref[...], preferred_element_type=jnp.float32)
    o_ref[...] = xs + dx


def _trunk0(p, pr):
    Bc, Tc, _ = p.shape
    p2 = p.reshape(Bc * Tc, DIMC)
    grid = (Bc * Tc) // _TBLK
    out = pl.pallas_call(
        _trunk0_body,
        grid=(grid,),
        in_specs=[
            pl.BlockSpec((_TBLK, DIMC), lambda i: (i, 0)),
            pl.BlockSpec((DIMC, HID2), lambda i: (0, 0)),
            pl.BlockSpec((HID2,), lambda i: (0,)),
            pl.BlockSpec((HID2, HID), lambda i: (0, 0)),
            pl.BlockSpec((HID,), lambda i: (0,)),
            pl.BlockSpec((HID, HID), lambda i: (0, 0)),
            pl.BlockSpec((HID,), lambda i: (0,)),
            pl.BlockSpec((HID2, HID), lambda i: (0, 0)),
        ],
        out_specs=pl.BlockSpec((_TBLK, HID), lambda i: (i, 0)),
        out_shape=jax.ShapeDtypeStruct((Bc * Tc, HID), jnp.float32),
    )(p2, pr['fc_pos_w'], pr['fc_pos_b'], pr['blk0_fc0_w'], pr['blk0_fc0_b'],
      pr['blk0_fc1_w'], pr['blk0_fc1_b'], pr['blk0_sc_w'])
    return out.reshape(Bc, Tc, HID)


def kernel(p, params):
    Bc = p.shape[0]
    net_pl = _fc_planenet(p, params)
    plane_parameters = net_pl.reshape(Bc, -1, 3)
    C_mat = _change_basis(plane_parameters)
    L = C_mat.shape[1]
    net_pl_h = _lin(jax.nn.relu(net_pl), params['fc_ph_w'], params['fc_ph_b'])[:, None, :]
    indices = [_coord2index(_normalize_dyn(p, C_mat[:, l])) for l in range(L)]
    net = _trunk0(p, params)
    for i in range(1, N_BLOCKS):
        pooled = _pool_local(indices, net)
        net = jnp.concatenate([net, pooled], axis=2)
        net = _resblock(net, params, i)
    c = _lin(net, params['fc_c_w'], params['fc_c_b']) + net_pl_h
    feas = [_gen_plane_feat(p, c, C_mat[:, l]) for l in range(L)]
    return tuple(feas) + (C_mat,)


# speedup vs baseline: 1.3969x; 1.3969x over previous
import functools

import jax
import jax.numpy as jnp
import numpy as np
from jax import lax
from jax.experimental import pallas as pl
from jax.experimental.pallas import tpu as pltpu
from jax.experimental.pallas import tpu_sc as plsc

C_DIM = 32
HID = 32
DIM = 3
N_BLOCKS = 5
N_CH = 3
RESO = 64
PAD = 0.1


def _lin(x, w, b=None):
    y = x @ w
    return y + b if b is not None else y


def _fc_planenet(p, pr):
    net = _lin(p, pr['pl0_w'], pr['pl0_b'])
    net = jax.nn.relu(_lin(net, pr['pl1_w'], pr['pl1_b']))
    net = jax.nn.relu(_lin(net, pr['pl2_w'], pr['pl2_b']))
    net = jax.nn.relu(_lin(net, pr['pl3_w'], pr['pl3_b']))
    net = jnp.max(net, axis=1)
    net = jax.nn.relu(_lin(net, pr['pl4_w'], pr['pl4_b']))
    net = jax.nn.relu(_lin(net, pr['pl5_w'], pr['pl5_b']))
    net = _lin(net, pr['pl6_w'], pr['pl6_b'])
    return net


def _resblock(x, pr, i):
    net = _lin(jax.nn.relu(x), pr['blk%d_fc0_w' % i], pr['blk%d_fc0_b' % i])
    dx = _lin(jax.nn.relu(net), pr['blk%d_fc1_w' % i], pr['blk%d_fc1_b' % i])
    xs = x @ pr['blk%d_sc_w' % i]
    return xs + dx


def _change_basis(plane_parameters):
    Bc, L, _ = plane_parameters.shape
    n = Bc * L
    normal = plane_parameters.reshape(n, 3)
    normal = normal / jnp.linalg.norm(normal, axis=1, keepdims=True)
    normal = normal + 1e-06
    basis_x = jnp.tile(jnp.array([1.0, 0.0, 0.0], jnp.float32), (n, 1))
    basis_y = jnp.tile(jnp.array([0.0, 1.0, 0.0], jnp.float32), (n, 1))
    basis_z = jnp.tile(jnp.array([0.0, 0.0, 1.0], jnp.float32), (n, 1))
    v = jnp.cross(basis_z, normal)
    zero = jnp.zeros((n,), jnp.float32)
    row0 = jnp.stack([zero, -v[:, 2], v[:, 1]], axis=1)
    row1 = jnp.stack([v[:, 2], zero, -v[:, 0]], axis=1)
    row2 = jnp.stack([-v[:, 1], v[:, 0], zero], axis=1)
    skew = jnp.stack([row0, row1, row2], axis=1)
    idty = jnp.tile(jnp.eye(3, dtype=jnp.float32)[None], (n, 1, 1))
    dot = (1.0 - jnp.sum(normal * basis_z, axis=1))[:, None, None]
    div = (jnp.linalg.norm(v, axis=1) ** 2)[:, None, None]
    R = idty + skew + jnp.matmul(skew, skew) * dot / div
    new_x = jnp.matmul(R, basis_x[:, :, None])
    new_y = jnp.matmul(R, basis_y[:, :, None])
    new_z = jnp.matmul(R, basis_z[:, :, None])
    new_basis = jnp.concatenate([new_x, new_y, new_z], axis=2)
    C_inv = jnp.linalg.inv(new_basis)
    b_x = jnp.abs(new_x)[:, :, 0]
    b_y = jnp.abs(new_y)[:, :, 0]
    p_dummy = jnp.ones((n, 3), jnp.float32)
    p_x = (jnp.sum(b_x * p_dummy, axis=1, keepdims=True) / jnp.sum(b_x * b_x, axis=1, keepdims=True)) * b_x
    p_y = (jnp.sum(b_y * p_dummy, axis=1, keepdims=True) / jnp.sum(b_y * b_y, axis=1, keepdims=True)) * b_y
    c_x = jnp.linalg.norm(p_x, axis=1)
    c_y = jnp.linalg.norm(p_y, axis=1)
    normalizer = jnp.tile(jnp.maximum(c_x, c_y)[:, None, None], (1, 1, 3))
    C_mat = jnp.concatenate([C_inv, normalizer], axis=1).reshape(Bc, L, 4, 3)
    return C_mat


def _normalize_dyn(p, cmat):
    C_inv = cmat[:, :3, :]
    normalizer = cmat[:, 3, 0][:, None, None]
    p_new = jnp.einsum('bij,btj->bti', C_inv, p) / normalizer
    xy = p_new[:, :, :2]
    xy = xy / (1.0 + PAD + 1e-3) + 0.5
    xy = jnp.clip(xy, 0.0, 1.0 - 1e-6)
    return xy


def _coord2index(xy):
    x = jnp.clip((xy * RESO).astype(jnp.int32), 0, RESO - 1)
    return x[:, :, 0] + RESO * x[:, :, 1]


def _pool_local(indices, c):
    Bc, Tc, Fc = c.shape
    offs = (jnp.arange(Bc, dtype=jnp.int32) * RESO * RESO)[:, None]
    out = jnp.zeros_like(c)
    flat_c = c.reshape(Bc * Tc, Fc)
    for index in indices:
        idx_flat = (index + offs).reshape(-1)
        fea = jax.ops.segment_max(flat_c, idx_flat, num_segments=Bc * RESO * RESO)
        fea = jnp.where(jnp.isneginf(fea), 0.0, fea)
        out = out + fea[idx_flat].reshape(Bc, Tc, Fc)
    return out


def _gen_plane_feat(p, c, cmat):
    Bc, Tc, _ = p.shape
    Fc = c.shape[2]
    xy = _normalize_dyn(p, cmat)
    index = _coord2index(xy)
    offs = (jnp.arange(Bc, dtype=jnp.int32) * RESO * RESO)[:, None]
    idx_flat = (index + offs).reshape(-1)
    sums = jax.ops.segment_sum(c.reshape(Bc * Tc, Fc), idx_flat, num_segments=Bc * RESO * RESO)
    cnts = jax.ops.segment_sum(jnp.ones((Bc * Tc,), jnp.float32), idx_flat, num_segments=Bc * RESO * RESO)
    mean = sums / jnp.maximum(cnts, 1.0)[:, None]
    fea = mean.reshape(Bc, RESO * RESO, Fc).transpose(0, 2, 1).reshape(Bc, Fc, RESO, RESO)
    return fea


_NSEG = RESO * RESO


def _sc_pool_max(net_h_flat, idx_flat, Bc, Tc, L=3, interpret=False):
    ncombo = L * Bc * 2
    per_core = ncombo // 2
    pch = 2000
    nch = Tc // pch
    mesh = plsc.VectorSubcoreMesh(core_axis_name="c", subcore_axis_name="s")

    @functools.partial(
        pl.kernel, mesh=mesh,
        out_type=jax.ShapeDtypeStruct((L * Bc * 2 * Tc * 16,), jnp.float32),
        scratch_types=[
            pltpu.VMEM((_NSEG * 16,), jnp.float32),
            pltpu.VMEM((pch,), jnp.int32),
            pltpu.VMEM((pch * 16,), jnp.float32),
        ],
        interpret=interpret,
    )
    def k(net_hbm, idx_hbm, out_hbm, grid_v, idx_v, in_v):
        cid = lax.axis_index("c")
        sid = lax.axis_index("s")

        @pl.when(sid < per_core)
        def _():
            combo = cid * per_core + sid
            l = combo // (Bc * 2)
            b = (combo // 2) % Bc
            h = combo % 2
            idx_base = (l * Bc + b) * Tc
            net_base = (b * 2 + h) * Tc * 16
            out_base = combo * Tc * 16
            neg = jnp.float32(-jnp.inf)

            def init(s, carry):
                grid_v[pl.ds(s * 16, 16)] = jnp.full((16,), neg, jnp.float32)
                return carry
            lax.fori_loop(0, _NSEG, init, 0)

            def chunk(ci, carry):
                c0 = ci * pch
                pltpu.sync_copy(
                    idx_hbm.at[pl.ds(pl.multiple_of(idx_base + c0, 8), pch)],
                    idx_v)
                pltpu.sync_copy(
                    net_hbm.at[pl.ds(pl.multiple_of(net_base + c0 * 16, 8),
                                     pch * 16)],
                    in_v)

                def pt(g, cc):
                    base = g * 16
                    idxvec = idx_v[pl.ds(base, 16)]
                    for j in range(16):
                        o = idxvec[j] * 16
                        row = in_v[pl.ds((base + j) * 16, 16)]
                        grid_v[pl.ds(o, 16)] = jnp.maximum(
                            grid_v[pl.ds(o, 16)], row)
                    return cc
                lax.fori_loop(0, pch // 16, pt, 0)
                return carry
            lax.fori_loop(0, nch, chunk, 0)

            def fix(s, carry):
                g = grid_v[pl.ds(s * 16, 16)]
                grid_v[pl.ds(s * 16, 16)] = jnp.where(
                    g == neg, jnp.float32(0.0), g)
                return carry
            lax.fori_loop(0, _NSEG, fix, 0)

            def chunk2(ci, carry):
                c0 = ci * pch
                pltpu.sync_copy(
                    idx_hbm.at[pl.ds(pl.multiple_of(idx_base + c0, 8), pch)],
                    idx_v)

                def pt(g, cc):
                    base = g * 16
                    idxvec = idx_v[pl.ds(base, 16)]
                    for j in range(16):
                        in_v[pl.ds((base + j) * 16, 16)] = grid_v[
                            pl.ds(idxvec[j] * 16, 16)]
                    return cc
                lax.fori_loop(0, pch // 16, pt, 0)
                pltpu.sync_copy(
                    in_v,
                    out_hbm.at[pl.ds(pl.multiple_of(out_base + c0 * 16, 8),
                                     pch * 16)])
                return carry
            lax.fori_loop(0, nch, chunk2, 0)

    return k(net_h_flat, idx_flat)


def _sc_pool_mean(c_h_flat, idx_flat, Bc, Tc, L=3, interpret=False):
    ncombo = L * Bc * 2
    per_core = ncombo // 2
    ncnt = L * Bc
    cnt_per_core = ncnt // 2
    pch = 2000
    nch = Tc // pch
    mesh = plsc.VectorSubcoreMesh(core_axis_name="c", subcore_axis_name="s")

    @functools.partial(
        pl.kernel, mesh=mesh,
        out_type=(jax.ShapeDtypeStruct((ncombo * _NSEG * 16,), jnp.float32),
                  jax.ShapeDtypeStruct((ncnt * _NSEG * 16,), jnp.float32)),
        scratch_types=[
            pltpu.VMEM((_NSEG * 16,), jnp.float32),
            pltpu.VMEM((pch,), jnp.int32),
            pltpu.VMEM((pch * 16,), jnp.float32),
        ],
        interpret=interpret,
    )
    def k(c_hbm, idx_hbm, sum_hbm, cnt_hbm, grid_v, idx_v, in_v):
        cid = lax.axis_index("c")
        sid = lax.axis_index("s")

        @pl.when(sid < per_core)
        def _():
            combo = cid * per_core + sid
            b = (combo // 2) % Bc
            h = combo % 2
            idx_base = (combo // 2) * Tc
            net_base = (b * 2 + h) * Tc * 16
            out_base = combo * _NSEG * 16

            def init(s, carry):
                grid_v[pl.ds(s * 16, 16)] = jnp.zeros((16,), jnp.float32)
                return carry
            lax.fori_loop(0, _NSEG, init, 0)

            def chunk(ci, carry):
                c0 = ci * pch
                pltpu.sync_copy(
                    idx_hbm.at[pl.ds(pl.multiple_of(idx_base + c0, 8), pch)],
                    idx_v)
                pltpu.sync_copy(
                    c_hbm.at[pl.ds(pl.multiple_of(net_base + c0 * 16, 8),
                                   pch * 16)],
                    in_v)

                def pt(g, cc):
                    base = g * 16
                    idxvec = idx_v[pl.ds(base, 16)]
                    for j in range(16):
                        o = idxvec[j] * 16
                        row = in_v[pl.ds((base + j) * 16, 16)]
                        grid_v[pl.ds(o, 16)] = grid_v[pl.ds(o, 16)] + row
                    return cc
                lax.fori_loop(0, pch // 16, pt, 0)
                return carry
            lax.fori_loop(0, nch, chunk, 0)

            pltpu.sync_copy(
                grid_v,
                sum_hbm.at[pl.ds(pl.multiple_of(out_base, 8), _NSEG * 16)])

        @pl.when(jnp.logical_and(sid >= per_core, sid < per_core + 3))
        def _():
            k0 = (sid - per_core) * 2
            ones = jnp.ones((16,), jnp.float32)
            for dk in range(2):
                task = cid * cnt_per_core + k0 + dk
                idx_base = task * Tc
                cbase = task * _NSEG * 16

                def initc(s, carry):
                    grid_v[pl.ds(s * 16, 16)] = jnp.zeros((16,), jnp.float32)
                    return carry
                lax.fori_loop(0, _NSEG, initc, 0)

                def chunkc(ci, carry):
                    c0 = ci * pch
                    pltpu.sync_copy(
                        idx_hbm.at[pl.ds(pl.multiple_of(idx_base + c0, 8),
                                         pch)], idx_v)

                    def ptc(g, cc):
                        idxvec = idx_v[pl.ds(g * 16, 16)]
                        for j in range(16):
                            o = idxvec[j] * 16
                            grid_v[pl.ds(o, 16)] = grid_v[pl.ds(o, 16)] + ones
                        return cc
                    lax.fori_loop(0, pch // 16, ptc, 0)
                    return carry
                lax.fori_loop(0, nch, chunkc, 0)

                pltpu.sync_copy(
                    grid_v,
                    cnt_hbm.at[pl.ds(pl.multiple_of(cbase, 8), _NSEG * 16)])

    return k(c_h_flat, idx_flat)


_PNB = 2048
_NPNB = 25


def _planenet_body(pt_ref, w0_ref, b0_ref, w1_ref, b1_ref, w2_ref, b2_ref,
                   w3_ref, b3_ref, o_ref):
    i = pl.program_id(0)
    Tc = 50000
    col = jax.lax.broadcasted_iota(jnp.int32, (1, _PNB), 1) + i * _PNB
    ok = col < Tc
    rows = []
    for bb in range(4):
        x = pt_ref[bb]
        h = jnp.dot(w0_ref[...], x, preferred_element_type=jnp.float32) + b0_ref[...]
        h = jax.nn.relu(jnp.dot(w1_ref[...], h, preferred_element_type=jnp.float32) + b1_ref[...])
        h = jax.nn.relu(jnp.dot(w2_ref[...], h, preferred_element_type=jnp.float32) + b2_ref[...])
        h = jax.nn.relu(jnp.dot(w3_ref[...], h, preferred_element_type=jnp.float32) + b3_ref[...])
        h = jnp.where(ok, h, -jnp.inf)
        rows.append(jnp.max(h, axis=1))
    m = jnp.stack(rows)

    @pl.when(i == 0)
    def _():
        o_ref[...] = m

    @pl.when(i > 0)
    def _():
        o_ref[...] = jnp.maximum(o_ref[...], m)


def _planenet_tc(p, pr):
    Bc, Tc, _ = p.shape
    p_t = jnp.transpose(p, (0, 2, 1))
    p_t = jnp.pad(p_t, ((0, 0), (0, 0), (0, _PNB * _NPNB - Tc)))
    wts = []
    for j in range(4):
        wts.append(jnp.transpose(pr['pl%d_w' % j]))
        wts.append(pr['pl%d_b' % j][:, None])
    out = pl.pallas_call(
        _planenet_body,
        grid=(_NPNB,),
        in_specs=[pl.BlockSpec((Bc, 3, _PNB), lambda i: (0, 0, i))] + [
            pl.BlockSpec(w.shape, lambda i: tuple([0] * w.ndim))
            for w in wts],
        out_specs=pl.BlockSpec((Bc, HID), lambda i: (0, 0)),
        out_shape=jax.ShapeDtypeStruct((Bc, HID), jnp.float32),
    )(p_t, *wts)
    net = jax.nn.relu(_lin(out, pr['pl4_w'], pr['pl4_b']))
    net = jax.nn.relu(_lin(net, pr['pl5_w'], pr['pl5_b']))
    net = _lin(net, pr['pl6_w'], pr['pl6_b'])
    return net


def _index_body(pt_ref, cm_ref, o_ref):
    cm = cm_ref[0, 0]
    cinv = cm[:3, :]
    norm = cm[3, 0]
    x = pt_ref[0]
    pn = jnp.dot(cinv, x, preferred_element_type=jnp.float32) / norm
    xy = pn[:2, :] / (1.0 + PAD + 1e-3) + 0.5
    xy = jnp.clip(xy, 0.0, 1.0 - 1e-6)
    xi = jnp.clip((xy * RESO).astype(jnp.int32), 0, RESO - 1)
    o_ref[0, 0, :] = xi[0, :] + RESO * xi[1, :]


def _index_tc(p_t_pad, C_mat):
    Bc = p_t_pad.shape[0]
    L = C_mat.shape[1]
    Tp = p_t_pad.shape[2]
    out = pl.pallas_call(
        _index_body,
        grid=(L, Bc, _NPNB),
        in_specs=[
            pl.BlockSpec((1, 3, _PNB), lambda l, b, i: (b, 0, i)),
            pl.BlockSpec((1, 1, 4, 3), lambda l, b, i: (b, l, 0, 0)),
        ],
        out_specs=pl.BlockSpec((1, 1, _PNB),
                               lambda l, b, i, Bc=Bc: (l * Bc + b, 0, i)),
        out_shape=jax.ShapeDtypeStruct((L * Bc, 1, Tp), jnp.int32),
    )(p_t_pad, C_mat)
    return out


_TBLK = 2000
HID2 = 2 * HID
DIMC = DIM


def _trunk0_body(p_ref, wpos_ref, bpos_ref, w00_ref, b00_ref, w01_ref,
                 b01_ref, wsc_ref, o_ref, o2_ref):
    p = p_ref[...]
    x = jnp.dot(p, wpos_ref[...], preferred_element_type=jnp.float32) + bpos_ref[...]
    net = jnp.dot(jax.nn.relu(x), w00_ref[...], preferred_element_type=jnp.float32) + b00_ref[...]
    dx = jnp.dot(jax.nn.relu(net), w01_ref[...], preferred_element_type=jnp.float32) + b01_ref[...]
    xs = jnp.dot(x, wsc_ref[...], preferred_element_type=jnp.float32)
    y = xs + dx
    o_ref[...] = y
    o2_ref[0, 0] = y[:, :16]
    o2_ref[0, 1] = y[:, 16:]


def _trunk0(p, pr):
    Bc, Tc, _ = p.shape
    p2 = p.reshape(Bc * Tc, DIMC)
    grid = (Bc * Tc) // _TBLK
    nb = Tc // _TBLK
    out = pl.pallas_call(
        _trunk0_body,
        grid=(grid,),
        in_specs=[
            pl.BlockSpec((_TBLK, DIMC), lambda i: (i, 0)),
            pl.BlockSpec((DIMC, HID2), lambda i: (0, 0)),
            pl.BlockSpec((HID2,), lambda i: (0,)),
            pl.BlockSpec((HID2, HID), lambda i: (0, 0)),
            pl.BlockSpec((HID,), lambda i: (0,)),
            pl.BlockSpec((HID, HID), lambda i: (0, 0)),
            pl.BlockSpec((HID,), lambda i: (0,)),
            pl.BlockSpec((HID2, HID), lambda i: (0, 0)),
        ],
        out_specs=[
            pl.BlockSpec((_TBLK, HID), lambda i: (i, 0)),
            pl.BlockSpec((1, 2, _TBLK, 16),
                         lambda i, nb=nb: (i // nb, 0, i % nb, 0)),
        ],
        out_shape=[
            jax.ShapeDtypeStruct((Bc * Tc, HID), jnp.float32),
            jax.ShapeDtypeStruct((Bc, 2, Tc, 16), jnp.float32),
        ],
    )(p2, pr['fc_pos_w'], pr['fc_pos_b'], pr['blk0_fc0_w'], pr['blk0_fc0_b'],
      pr['blk0_fc1_w'], pr['blk0_fc1_b'], pr['blk0_sc_w'])
    return out[0], out[1].reshape(-1)


def _resblock_mid_body(net_ref, parts_ref, w0_ref, b0_ref, w1_ref, b1_ref,
                       wsc_ref, o1_ref, o2_ref):
    x_in = net_ref[...]
    pv = parts_ref[...]
    pooled = pv[0] + pv[1] + pv[2]
    pooled32 = jnp.concatenate([pooled[0, 0], pooled[0, 1]], axis=1)
    x = jnp.concatenate([x_in, pooled32], axis=1)
    h = jnp.dot(jax.nn.relu(x), w0_ref[...], preferred_element_type=jnp.float32) + b0_ref[...]
    dx = jnp.dot(jax.nn.relu(h), w1_ref[...], preferred_element_type=jnp.float32) + b1_ref[...]
    xs = jnp.dot(x, wsc_ref[...], preferred_element_type=jnp.float32)
    y = xs + dx
    o1_ref[...] = y
    o2_ref[0, 0] = y[:, :16]
    o2_ref[0, 1] = y[:, 16:]


def _resblock_fin_body(net_ref, parts_ref, w0_ref, b0_ref, w1_ref, b1_ref,
                       wsc_ref, wc_ref, bc_ref, ph_ref, o2_ref):
    x_in = net_ref[...]
    pv = parts_ref[...]
    pooled = pv[0] + pv[1] + pv[2]
    pooled32 = jnp.concatenate([pooled[0, 0], pooled[0, 1]], axis=1)
    x = jnp.concatenate([x_in, pooled32], axis=1)
    h = jnp.dot(jax.nn.relu(x), w0_ref[...], preferred_element_type=jnp.float32) + b0_ref[...]
    dx = jnp.dot(jax.nn.relu(h), w1_ref[...], preferred_element_type=jnp.float32) + b1_ref[...]
    xs = jnp.dot(x, wsc_ref[...], preferred_element_type=jnp.float32)
    y = xs + dx
    c = jnp.dot(y, wc_ref[...], preferred_element_type=jnp.float32) + bc_ref[...] + ph_ref[0, 0]
    o2_ref[0, 0] = c[:, :16]
    o2_ref[0, 1] = c[:, 16:]


def _resblock_tc(net2d, parts, pr, i, Bc, Tc):
    nb = Tc // _TBLK
    grid = (Bc * Tc) // _TBLK
    out = pl.pallas_call(
        _resblock_mid_body,
        grid=(grid,),
        in_specs=[
            pl.BlockSpec((_TBLK, HID), lambda i: (i, 0)),
            pl.BlockSpec((3, 1, 2, _TBLK, 16),
                         lambda i, nb=nb: (0, i // nb, 0, i % nb, 0)),
            pl.BlockSpec((HID2, HID), lambda i: (0, 0)),
            pl.BlockSpec((HID,), lambda i: (0,)),
            pl.BlockSpec((HID, HID), lambda i: (0, 0)),
            pl.BlockSpec((HID,), lambda i: (0,)),
            pl.BlockSpec((HID2, HID), lambda i: (0, 0)),
        ],
        out_specs=[
            pl.BlockSpec((_TBLK, HID), lambda i: (i, 0)),
            pl.BlockSpec((1, 2, _TBLK, 16),
                         lambda i, nb=nb: (i // nb, 0, i % nb, 0)),
        ],
        out_shape=[
            jax.ShapeDtypeStruct((Bc * Tc, HID), jnp.float32),
            jax.ShapeDtypeStruct((Bc, 2, Tc, 16), jnp.float32),
        ],
    )(net2d, parts, pr['blk%d_fc0_w' % i], pr['blk%d_fc0_b' % i],
      pr['blk%d_fc1_w' % i], pr['blk%d_fc1_b' % i], pr['blk%d_sc_w' % i])
    return out[0], out[1].reshape(-1)


def _resblock_fin_tc(net2d, parts, pr, net_pl_h, Bc, Tc):
    nb = Tc // _TBLK
    grid = (Bc * Tc) // _TBLK
    i = N_BLOCKS - 1
    out = pl.pallas_call(
        _resblock_fin_body,
        grid=(grid,),
        in_specs=[
            pl.BlockSpec((_TBLK, HID), lambda i: (i, 0)),
            pl.BlockSpec((3, 1, 2, _TBLK, 16),
                         lambda i, nb=nb: (0, i // nb, 0, i % nb, 0)),
            pl.BlockSpec((HID2, HID), lambda i: (0, 0)),
            pl.BlockSpec((HID,), lambda i: (0,)),
            pl.BlockSpec((HID, HID), lambda i: (0, 0)),
            pl.BlockSpec((HID,), lambda i: (0,)),
            pl.BlockSpec((HID2, HID), lambda i: (0, 0)),
            pl.BlockSpec((HID, C_DIM), lambda i: (0, 0)),
            pl.BlockSpec((C_DIM,), lambda i: (0,)),
            pl.BlockSpec((1, 1, C_DIM), lambda i, nb=nb: (i // nb, 0, 0)),
        ],
        out_specs=pl.BlockSpec((1, 2, _TBLK, 16),
                               lambda i, nb=nb: (i // nb, 0, i % nb, 0)),
        out_shape=jax.ShapeDtypeStruct((Bc, 2, Tc, 16), jnp.float32),
    )(net2d, parts, pr['blk%d_fc0_w' % i], pr['blk%d_fc0_b' % i],
      pr['blk%d_fc1_w' % i], pr['blk%d_fc1_b' % i], pr['blk%d_sc_w' % i],
      pr['fc_c_w'], pr['fc_c_b'], net_pl_h[:, None, :])
    return out.reshape(-1)


def kernel(p, params):
    Bc, Tc, _ = p.shape
    L = N_CH
    net_pl = _planenet_tc(p, params)
    C_mat = _change_basis(net_pl.reshape(Bc, -1, 3))
    net_pl_h = _lin(jax.nn.relu(net_pl),
                    params['fc_ph_w'], params['fc_ph_b'])
    p_t_pad = jnp.pad(jnp.transpose(p, (0, 2, 1)),
                      ((0, 0), (0, 0), (0, _PNB * _NPNB - Tc)))
    idx_pad = _index_tc(p_t_pad, C_mat)
    idx_flat = idx_pad[:, 0, :Tc].reshape(-1)
    net2d, net_h = _trunk0(p, params)
    for i in range(1, N_BLOCKS - 1):
        parts = _sc_pool_max(net_h, idx_flat, Bc, Tc)
        parts = parts.reshape(L, Bc, 2, Tc, 16)
        net2d, net_h = _resblock_tc(net2d, parts, params, i, Bc, Tc)
    parts = _sc_pool_max(net_h, idx_flat, Bc, Tc)
    parts = parts.reshape(L, Bc, 2, Tc, 16)
    c_h = _resblock_fin_tc(net2d, parts, params, net_pl_h, Bc, Tc)
    sums, cnts = _sc_pool_mean(c_h, idx_flat, Bc, Tc)
    sums = sums.reshape(L, Bc, 2, _NSEG, 16)
    cntf = cnts.reshape(L, Bc, _NSEG, 16)[..., 0]
    mean = sums / jnp.maximum(cntf, 1.0)[:, :, None, :, None]
    feas = [mean[l].transpose(0, 1, 3, 2).reshape(Bc, 2 * 16, RESO, RESO)
            for l in range(L)]
    return tuple(feas) + (C_mat,)
